# baseline (speedup 1.0000x reference)
import jax
import jax.numpy as jnp
from jax import lax
from jax.experimental import pallas as pl
from jax.experimental.pallas import tpu as pltpu

_C = 16384
_D = 512
_M = 1024
_ALPHA = 0.2
_BC = 1024


_CHUNK = 256


def _fused_kernel(x_ref, cache_ref, w_ref, out_ref):
    cache = cache_ref[...]
    w = w_ref[...]
    cb = cache.astype(jnp.bfloat16)
    wb = w.astype(jnp.bfloat16)
    w1 = wb[:, :_D]
    w2 = wb[:, _D:]

    for k in range(_BC // _CHUNK):
        rows = pl.ds(k * _CHUNK, _CHUNK)
        x = x_ref[rows, :]
        xb = x.astype(jnp.bfloat16)
        s = lax.dot_general(xb, cb, (((1,), (1,)), ((), ())),
                            preferred_element_type=jnp.float32)
        e = jnp.exp(s)
        denom = jnp.sum(e, axis=1, keepdims=True)
        f = lax.dot_general(e.astype(jnp.bfloat16), cb, (((1,), (0,)), ((), ())),
                            preferred_element_type=jnp.float32) / denom
        p1 = lax.dot_general(xb, w1, (((1,), (1,)), ((), ())),
                             preferred_element_type=jnp.float32)
        p2 = lax.dot_general(f.astype(jnp.bfloat16), w2, (((1,), (1,)), ((), ())),
                             preferred_element_type=jnp.float32)
        out_ref[rows, :] = _ALPHA * (p1 + p2) + x


@jax.jit
def _run(text_token, cache, W):
    grid = (_C // _BC,)
    return pl.pallas_call(
        _fused_kernel,
        grid=grid,
        in_specs=[
            pl.BlockSpec((_BC, _D), lambda i: (i, 0)),
            pl.BlockSpec((_M, _D), lambda i: (0, 0)),
            pl.BlockSpec((_D, 2 * _D), lambda i: (0, 0)),
        ],
        out_specs=pl.BlockSpec((_BC, _D), lambda i: (i, 0)),
        out_shape=jax.ShapeDtypeStruct((_C, _D), jnp.float32),
        compiler_params=pltpu.CompilerParams(
            dimension_semantics=("arbitrary",),
        ),
    )(text_token, cache, W)


def kernel(text_token, image_token, cache, W):
    out = _run(text_token, cache, W)
    return (out, jnp.float32(0.0))


# speedup vs baseline: 1.0258x; 1.0258x over previous
import jax
import jax.numpy as jnp
from jax import lax
from jax.experimental import pallas as pl
from jax.experimental.pallas import tpu as pltpu

_C = 16384
_D = 512
_M = 1024
_ALPHA = 0.2
_BC = 512


_CHUNK = 512


def _fused_kernel(x_ref, cache_ref, w_ref, out_ref):
    cache = cache_ref[...]
    w = w_ref[...]
    cb = cache.astype(jnp.bfloat16)
    wb = w.astype(jnp.bfloat16)
    w1 = wb[:, :_D]
    w2 = wb[:, _D:]

    for k in range(_BC // _CHUNK):
        rows = pl.ds(k * _CHUNK, _CHUNK)
        x = x_ref[rows, :]
        xb = x.astype(jnp.bfloat16)
        s = lax.dot_general(xb, cb, (((1,), (1,)), ((), ())),
                            preferred_element_type=jnp.float32)
        e = jnp.exp(s)
        denom = jnp.sum(e, axis=1, keepdims=True)
        f = lax.dot_general(e.astype(jnp.bfloat16), cb, (((1,), (0,)), ((), ())),
                            preferred_element_type=jnp.float32) / denom
        p1 = lax.dot_general(xb, w1, (((1,), (1,)), ((), ())),
                             preferred_element_type=jnp.float32)
        p2 = lax.dot_general(f.astype(jnp.bfloat16), w2, (((1,), (1,)), ((), ())),
                             preferred_element_type=jnp.float32)
        out_ref[rows, :] = _ALPHA * (p1 + p2) + x


@jax.jit
def _run(text_token, cache, W):
    grid = (_C // _BC,)
    return pl.pallas_call(
        _fused_kernel,
        grid=grid,
        in_specs=[
            pl.BlockSpec((_BC, _D), lambda i: (i, 0)),
            pl.BlockSpec((_M, _D), lambda i: (0, 0)),
            pl.BlockSpec((_D, 2 * _D), lambda i: (0, 0)),
        ],
        out_specs=pl.BlockSpec((_BC, _D), lambda i: (i, 0)),
        out_shape=jax.ShapeDtypeStruct((_C, _D), jnp.float32),
        compiler_params=pltpu.CompilerParams(
            dimension_semantics=("arbitrary",),
        ),
    )(text_token, cache, W)


def kernel(text_token, image_token, cache, W):
    out = _run(text_token, cache, W)
    return (out, jnp.float32(0.0))


# speedup vs baseline: 1.1396x; 1.1110x over previous
import jax
import jax.numpy as jnp
from jax import lax
from jax.experimental import pallas as pl
from jax.experimental.pallas import tpu as pltpu

_C = 16384
_D = 512
_M = 1024
_ALPHA = 0.2
_BC = 1024


_CHUNK = 1024


def _fused_kernel(x_ref, cache_ref, w_ref, out_ref):
    cache = cache_ref[...]
    w = w_ref[...]
    cb = cache.astype(jnp.bfloat16)
    wb = w.astype(jnp.bfloat16)
    w1 = wb[:, :_D]
    w2 = wb[:, _D:]

    for k in range(_BC // _CHUNK):
        rows = pl.ds(k * _CHUNK, _CHUNK)
        x = x_ref[rows, :]
        xb = x.astype(jnp.bfloat16)
        s = lax.dot_general(xb, cb, (((1,), (1,)), ((), ())),
                            preferred_element_type=jnp.float32)
        e = jnp.exp(s)
        denom = jnp.sum(e, axis=1, keepdims=True)
        f = lax.dot_general(e.astype(jnp.bfloat16), cb, (((1,), (0,)), ((), ())),
                            preferred_element_type=jnp.float32) / denom
        p1 = lax.dot_general(xb, w1, (((1,), (1,)), ((), ())),
                             preferred_element_type=jnp.float32)
        p2 = lax.dot_general(f.astype(jnp.bfloat16), w2, (((1,), (1,)), ((), ())),
                             preferred_element_type=jnp.float32)
        out_ref[rows, :] = _ALPHA * (p1 + p2) + x


@jax.jit
def _run(text_token, cache, W):
    grid = (_C // _BC,)
    return pl.pallas_call(
        _fused_kernel,
        grid=grid,
        in_specs=[
            pl.BlockSpec((_BC, _D), lambda i: (i, 0)),
            pl.BlockSpec((_M, _D), lambda i: (0, 0)),
            pl.BlockSpec((_D, 2 * _D), lambda i: (0, 0)),
        ],
        out_specs=pl.BlockSpec((_BC, _D), lambda i: (i, 0)),
        out_shape=jax.ShapeDtypeStruct((_C, _D), jnp.float32),
        compiler_params=pltpu.CompilerParams(
            dimension_semantics=("arbitrary",),
        ),
    )(text_token, cache, W)


def kernel(text_token, image_token, cache, W):
    out = _run(text_token, cache, W)
    return (out, jnp.float32(0.0))
